# baseline (speedup 1.0000x reference)
import functools

import jax
import jax.numpy as jnp
from jax import lax
from jax.experimental import pallas as pl
from jax.experimental.pallas import tpu as pltpu
from jax.experimental.pallas import tpu_sc as plsc

N = 10000
E = 320000
D = 128
NC = 2
NS = 16
NW = NC * NS
K = 80
CPW = E // K // NW
RPT = 632
RPT_LAST = N - (NS - 1) * RPT
DW = 128
NSUB = 5
KH = K // NSUB
BLK = 1000
GRID = N // BLK

_mesh = plsc.VectorSubcoreMesh(
    core_axis_name="c", subcore_axis_name="s", num_cores=NC, num_subcores=NS)


def _degree_body(dst_hbm, zeros1_hbm, ones_hbm, out_hbm, deg_sp, dst_v, ones_v,
                 sem):
    c = lax.axis_index("c")
    s = lax.axis_index("s")
    wid = s * NC + c

    @pl.when(s == 0)
    def _():
        pltpu.sync_copy(zeros1_hbm, deg_sp)

    pltpu.sync_copy(dst_hbm.at[wid], dst_v)
    pltpu.sync_copy(ones_hbm, ones_v)
    plsc.subcore_barrier()

    FL = 5

    def body(i, carry):
        for u in range(FL):
            pltpu.async_copy(ones_v, deg_sp.at[dst_v.at[i * FL + u]], sem,
                             add=True)
        for u in range(FL):
            pltpu.make_async_copy(ones_v, deg_sp.at[dst_v.at[i * FL + u]],
                                  sem).wait()
        return carry

    lax.fori_loop(0, CPW // FL, body, 0)
    plsc.subcore_barrier()

    @pl.when(s == 0)
    def _():
        pltpu.sync_copy(deg_sp, out_hbm.at[pl.ds(c * N, N)])


def _gather_scatter_body(g_hbm, src_hbm, dstf_hbm, zeros_hbm, out_hbm,
                         acc_sp, src_v, dst_st0, dst_st1, rows0_v, rows1_v,
                         sem0, sem1, semd0, semd1):
    c = lax.axis_index("c")
    s = lax.axis_index("s")
    wid = s * NC + c
    ebase = wid * CPW * K

    @pl.when(s < NS - 1)
    def _():
        pltpu.sync_copy(zeros_hbm.at[pl.ds(s * RPT, RPT)],
                        acc_sp.at[pl.ds(s * RPT, RPT)])

    @pl.when(s == NS - 1)
    def _():
        pltpu.sync_copy(zeros_hbm.at[pl.ds((NS - 1) * RPT, RPT_LAST)],
                        acc_sp.at[pl.ds((NS - 1) * RPT, RPT_LAST)])

    pltpu.sync_copy(src_hbm.at[wid], src_v)
    plsc.subcore_barrier()

    def dst_load(j, st, sem):
        pltpu.async_copy(dstf_hbm.at[pl.ds(ebase + j * K, K)], st, sem)

    def dst_wait(j, st, sem):
        pltpu.make_async_copy(dstf_hbm.at[pl.ds(ebase + j * K, K)], st,
                              sem).wait()

    def g_issue(j, rows, sem):
        for u in range(NSUB):
            pltpu.async_copy(g_hbm.at[src_v.at[j, pl.ds(u * KH, KH)]],
                             rows.at[pl.ds(u * KH, KH)], sem)

    def g_wait(j, rows, sem):
        for u in range(NSUB):
            pltpu.make_async_copy(g_hbm.at[src_v.at[j, pl.ds(u * KH, KH)]],
                                  rows.at[pl.ds(u * KH, KH)], sem).wait()

    dst_load(0, dst_st0, semd0)
    dst_load(1, dst_st1, semd1)
    g_issue(0, rows0_v, sem0)

    def body2(i, carry):
        j0 = 2 * i
        j1 = j0 + 1
        g_wait(j0, rows0_v, sem0)
        g_issue(j1, rows1_v, sem1)
        dst_wait(j0, dst_st0, semd0)
        pltpu.sync_copy(rows0_v, acc_sp.at[dst_st0], add=True)

        @pl.when(j0 + 2 < CPW)
        def _():
            dst_load(j0 + 2, dst_st0, semd0)

        g_wait(j1, rows1_v, sem1)

        @pl.when(j0 + 2 < CPW)
        def _():
            g_issue(j0 + 2, rows0_v, sem0)

        dst_wait(j1, dst_st1, semd1)
        pltpu.sync_copy(rows1_v, acc_sp.at[dst_st1], add=True)

        @pl.when(j1 + 2 < CPW)
        def _():
            dst_load(j1 + 2, dst_st1, semd1)

        return carry

    lax.fori_loop(0, CPW // 2, body2, 0)
    g_wait(CPW - 1, rows0_v, sem0)
    dst_wait(CPW - 1, dst_st0, semd0)
    pltpu.sync_copy(rows0_v, acc_sp.at[dst_st0], add=True)
    plsc.subcore_barrier()

    @pl.when(s < NS - 1)
    def _():
        pltpu.sync_copy(acc_sp.at[pl.ds(s * RPT, RPT)],
                        out_hbm.at[pl.ds(c * N + s * RPT, RPT)])

    @pl.when(s == NS - 1)
    def _():
        pltpu.sync_copy(acc_sp.at[pl.ds((NS - 1) * RPT, RPT_LAST)],
                        out_hbm.at[pl.ds(c * N + (NS - 1) * RPT, RPT_LAST)])


def _make_degree(interpret=False):
    return pl.kernel(
        _degree_body,
        out_type=jax.ShapeDtypeStruct((NC * N, DW), jnp.float32),
        mesh=_mesh,
        scratch_types=[
            pltpu.VMEM_SHARED((N, DW), jnp.float32),
            pltpu.VMEM((CPW, K), jnp.int32),
            pltpu.VMEM((K, DW), jnp.float32),
            pltpu.SemaphoreType.DMA,
        ],
        interpret=interpret,
    )


def _make_gather_scatter(interpret=False):
    return pl.kernel(
        _gather_scatter_body,
        out_type=jax.ShapeDtypeStruct((NC * N, D), jnp.float32),
        mesh=_mesh,
        scratch_types=[
            pltpu.VMEM_SHARED((N, D), jnp.float32),
            pltpu.VMEM((CPW, K), jnp.int32),
            pltpu.VMEM((K,), jnp.int32),
            pltpu.VMEM((K,), jnp.int32),
            pltpu.VMEM((K, D), jnp.float32),
            pltpu.VMEM((K, D), jnp.float32),
            pltpu.SemaphoreType.DMA,
            pltpu.SemaphoreType.DMA,
            pltpu.SemaphoreType.DMA,
            pltpu.SemaphoreType.DMA,
        ],
        interpret=interpret,
    )


_degree = _make_degree()
_gather_scatter = _make_gather_scatter()


def _tc0_body(x_ref, w_ref, dega_ref, degb_ref, dis_ref, g_ref):
    deg = dega_ref[:, 0:1] + degb_ref[:, 0:1] + 1.0
    dis = lax.rsqrt(deg)
    dis_ref[...] = dis
    g_ref[...] = jnp.dot(x_ref[...], w_ref[...],
                         preferred_element_type=jnp.float32) * dis


def _tc_mid_body(acca_ref, accb_ref, g_ref, dis_ref, b_ref, w_ref,
                 x_ref, gout_ref):
    dis = dis_ref[...]
    xl = dis * (acca_ref[...] + accb_ref[...] + g_ref[...]) + b_ref[...]
    x_ref[...] = xl
    gout_ref[...] = jnp.dot(xl, w_ref[...],
                            preferred_element_type=jnp.float32) * dis


def _tc_last_body(acca_ref, accb_ref, g_ref, dis_ref, b_ref, x_ref):
    x_ref[...] = (dis_ref[...] * (acca_ref[...] + accb_ref[...] + g_ref[...])
                  + b_ref[...])


def _rows_spec(shape):
    return pl.BlockSpec((BLK,) + shape[1:], lambda i: (i,) + (0,) * (len(shape) - 1))


def _rows_spec_hi(shape):
    return pl.BlockSpec((BLK,) + shape[1:],
                        lambda i: (i + GRID,) + (0,) * (len(shape) - 1))


_full_w = pl.BlockSpec((D, D), lambda i: (0, 0))
_full_b = pl.BlockSpec((1, D), lambda i: (0, 0))

_tc0 = pl.pallas_call(
    _tc0_body,
    grid=(GRID,),
    in_specs=[_rows_spec((N, D)), _full_w,
              _rows_spec((N, DW)), _rows_spec_hi((N, DW))],
    out_specs=[_rows_spec((N, 1)), _rows_spec((N, D))],
    out_shape=[jax.ShapeDtypeStruct((N, 1), jnp.float32),
               jax.ShapeDtypeStruct((N, D), jnp.float32)],
)

_tc_mid = pl.pallas_call(
    _tc_mid_body,
    grid=(GRID,),
    in_specs=[_rows_spec((N, D)), _rows_spec_hi((N, D)), _rows_spec((N, D)),
              _rows_spec((N, 1)), _full_b, _full_w],
    out_specs=[_rows_spec((N, D)), _rows_spec((N, D))],
    out_shape=[jax.ShapeDtypeStruct((N, D), jnp.float32),
               jax.ShapeDtypeStruct((N, D), jnp.float32)],
)

_tc_last = pl.pallas_call(
    _tc_last_body,
    grid=(GRID,),
    in_specs=[_rows_spec((N, D)), _rows_spec_hi((N, D)), _rows_spec((N, D)),
              _rows_spec((N, 1)), _full_b],
    out_specs=_rows_spec((N, D)),
    out_shape=jax.ShapeDtypeStruct((N, D), jnp.float32),
)


def kernel(x, edge_index, W0, b0, W1, b1, W2, b2):
    src = edge_index[0].astype(jnp.int32).reshape(NW, CPW, K)
    dstf = edge_index[1].astype(jnp.int32)
    dst = dstf.reshape(NW, CPW, K)
    zeros = jnp.zeros((N, D), jnp.float32)
    zeros1 = jnp.zeros((N, DW), jnp.float32)
    ones = jnp.ones((K, DW), jnp.float32)

    deg2 = _degree(dst, zeros1, ones)
    dis, g0 = _tc0(x, W0, deg2, deg2)

    b0 = b0.reshape(1, D)
    b1 = b1.reshape(1, D)
    b2 = b2.reshape(1, D)

    acc0 = _gather_scatter(g0, src, dstf, zeros)
    x1, g1 = _tc_mid(acc0, acc0, g0, dis, b0, W1)
    acc1 = _gather_scatter(g1, src, dstf, zeros)
    x2, g2 = _tc_mid(acc1, acc1, g1, dis, b1, W2)
    acc2 = _gather_scatter(g2, src, dstf, zeros)
    x3 = _tc_last(acc2, acc2, g2, dis, b2)
    return jnp.concatenate([x, x1, x2, x3], axis=-1)


# speedup vs baseline: 1.1083x; 1.1083x over previous
import functools

import jax
import jax.numpy as jnp
from jax import lax
from jax.experimental import pallas as pl
from jax.experimental.pallas import tpu as pltpu
from jax.experimental.pallas import tpu_sc as plsc

N = 10000
E = 320000
D = 128
NC = 2
NS = 16
NW = NC * NS
K = 80
CPW = E // K // NW
RPT = 632
RPT_LAST = N - (NS - 1) * RPT
DW = 128
NSUB = 2
NBUF = 3
KH = K // NSUB
BLK = 1000
GRID = N // BLK

_mesh = plsc.VectorSubcoreMesh(
    core_axis_name="c", subcore_axis_name="s", num_cores=NC, num_subcores=NS)


def _degree_body(dst_hbm, zeros1_hbm, ones_hbm, out_hbm, deg_sp, dst_v, ones_v,
                 sem):
    c = lax.axis_index("c")
    s = lax.axis_index("s")
    wid = s * NC + c

    @pl.when(s == 0)
    def _():
        pltpu.sync_copy(zeros1_hbm, deg_sp)

    pltpu.sync_copy(dst_hbm.at[wid], dst_v)
    pltpu.sync_copy(ones_hbm, ones_v)
    plsc.subcore_barrier()

    FL = 5

    def body(i, carry):
        for u in range(FL):
            pltpu.async_copy(ones_v, deg_sp.at[dst_v.at[i * FL + u]], sem,
                             add=True)
        for u in range(FL):
            pltpu.make_async_copy(ones_v, deg_sp.at[dst_v.at[i * FL + u]],
                                  sem).wait()
        return carry

    lax.fori_loop(0, CPW // FL, body, 0)
    plsc.subcore_barrier()

    @pl.when(s == 0)
    def _():
        pltpu.sync_copy(deg_sp, out_hbm.at[pl.ds(c * N, N)])


def _gather_scatter_body(g_hbm, srcf_hbm, dstf_hbm, zeros_hbm, out_hbm,
                         acc_sp, src_st, dst_st, rows_v, gsems, isems):
    c = lax.axis_index("c")
    s = lax.axis_index("s")
    wid = s * NC + c
    ebase = wid * CPW * K

    @pl.when(s < NS - 1)
    def _():
        pltpu.sync_copy(zeros_hbm.at[pl.ds(s * RPT, RPT)],
                        acc_sp.at[pl.ds(s * RPT, RPT)])

    @pl.when(s == NS - 1)
    def _():
        pltpu.sync_copy(zeros_hbm.at[pl.ds((NS - 1) * RPT, RPT_LAST)],
                        acc_sp.at[pl.ds((NS - 1) * RPT, RPT_LAST)])

    def idx_load(j, b):
        pltpu.async_copy(srcf_hbm.at[pl.ds(ebase + j * K, K)], src_st[b],
                         isems[b])
        pltpu.async_copy(dstf_hbm.at[pl.ds(ebase + j * K, K)], dst_st[b],
                         isems[b])

    def idx_wait(j, b):
        pltpu.make_async_copy(srcf_hbm.at[pl.ds(ebase + j * K, K)], src_st[b],
                              isems[b]).wait()
        pltpu.make_async_copy(dstf_hbm.at[pl.ds(ebase + j * K, K)], dst_st[b],
                              isems[b]).wait()

    def g_issue(b):
        for u in range(NSUB):
            pltpu.async_copy(g_hbm.at[src_st[b].at[pl.ds(u * KH, KH)]],
                             rows_v[b].at[pl.ds(u * KH, KH)], gsems[b])

    def g_wait(b):
        for u in range(NSUB):
            pltpu.make_async_copy(g_hbm.at[src_st[b].at[pl.ds(u * KH, KH)]],
                                  rows_v[b].at[pl.ds(u * KH, KH)],
                                  gsems[b]).wait()

    for b in range(NBUF):
        idx_load(b, b)
    plsc.subcore_barrier()
    for b in range(2):
        idx_wait(b, b)
        g_issue(b)

    def step(j, b):
        g_wait(b)
        b2 = (b + 2) % NBUF

        @pl.when(j + 2 < CPW)
        def _():
            idx_wait(j + 2, b2)
            g_issue(b2)

        pltpu.sync_copy(rows_v[b], acc_sp.at[dst_st[b]], add=True)

        @pl.when(j + NBUF < CPW)
        def _():
            idx_load(j + NBUF, b)

    def bodyn(i, carry):
        for u in range(NBUF):
            step(NBUF * i + u, u)
        return carry

    lax.fori_loop(0, CPW // NBUF, bodyn, 0)
    for u in range(CPW % NBUF):
        step((CPW // NBUF) * NBUF + u, u)
    plsc.subcore_barrier()

    @pl.when(s < NS - 1)
    def _():
        pltpu.sync_copy(acc_sp.at[pl.ds(s * RPT, RPT)],
                        out_hbm.at[pl.ds(c * N + s * RPT, RPT)])

    @pl.when(s == NS - 1)
    def _():
        pltpu.sync_copy(acc_sp.at[pl.ds((NS - 1) * RPT, RPT_LAST)],
                        out_hbm.at[pl.ds(c * N + (NS - 1) * RPT, RPT_LAST)])


def _make_degree(interpret=False):
    return pl.kernel(
        _degree_body,
        out_type=jax.ShapeDtypeStruct((NC * N, DW), jnp.float32),
        mesh=_mesh,
        scratch_types=[
            pltpu.VMEM_SHARED((N, DW), jnp.float32),
            pltpu.VMEM((CPW, K), jnp.int32),
            pltpu.VMEM((K, DW), jnp.float32),
            pltpu.SemaphoreType.DMA,
        ],
        interpret=interpret,
    )


def _make_gather_scatter(interpret=False):
    return pl.kernel(
        _gather_scatter_body,
        out_type=jax.ShapeDtypeStruct((NC * N, D), jnp.float32),
        mesh=_mesh,
        scratch_types=[
            pltpu.VMEM_SHARED((N, D), jnp.float32),
            [pltpu.VMEM((K,), jnp.int32) for _ in range(NBUF)],
            [pltpu.VMEM((K,), jnp.int32) for _ in range(NBUF)],
            [pltpu.VMEM((K, D), jnp.float32) for _ in range(NBUF)],
            [pltpu.SemaphoreType.DMA for _ in range(NBUF)],
            [pltpu.SemaphoreType.DMA for _ in range(NBUF)],
        ],
        interpret=interpret,
    )


_degree = _make_degree()
_gather_scatter = _make_gather_scatter()


def _tc0_body(x_ref, w_ref, dega_ref, degb_ref, dis_ref, g_ref):
    deg = dega_ref[:, 0:1] + degb_ref[:, 0:1] + 1.0
    dis = lax.rsqrt(deg)
    dis_ref[...] = dis
    g_ref[...] = jnp.dot(x_ref[...], w_ref[...],
                         preferred_element_type=jnp.float32) * dis


def _tc_mid_body(acca_ref, accb_ref, g_ref, dis_ref, b_ref, w_ref,
                 x_ref, gout_ref):
    dis = dis_ref[...]
    xl = dis * (acca_ref[...] + accb_ref[...] + g_ref[...]) + b_ref[...]
    x_ref[...] = xl
    gout_ref[...] = jnp.dot(xl, w_ref[...],
                            preferred_element_type=jnp.float32) * dis


def _tc_last_body(acca_ref, accb_ref, g_ref, dis_ref, b_ref, x_ref):
    x_ref[...] = (dis_ref[...] * (acca_ref[...] + accb_ref[...] + g_ref[...])
                  + b_ref[...])


def _rows_spec(shape):
    return pl.BlockSpec((BLK,) + shape[1:], lambda i: (i,) + (0,) * (len(shape) - 1))


def _rows_spec_hi(shape):
    return pl.BlockSpec((BLK,) + shape[1:],
                        lambda i: (i + GRID,) + (0,) * (len(shape) - 1))


_full_w = pl.BlockSpec((D, D), lambda i: (0, 0))
_full_b = pl.BlockSpec((1, D), lambda i: (0, 0))

_tc0 = pl.pallas_call(
    _tc0_body,
    grid=(GRID,),
    in_specs=[_rows_spec((N, D)), _full_w,
              _rows_spec((N, DW)), _rows_spec_hi((N, DW))],
    out_specs=[_rows_spec((N, 1)), _rows_spec((N, D))],
    out_shape=[jax.ShapeDtypeStruct((N, 1), jnp.float32),
               jax.ShapeDtypeStruct((N, D), jnp.float32)],
)

_tc_mid = pl.pallas_call(
    _tc_mid_body,
    grid=(GRID,),
    in_specs=[_rows_spec((N, D)), _rows_spec_hi((N, D)), _rows_spec((N, D)),
              _rows_spec((N, 1)), _full_b, _full_w],
    out_specs=[_rows_spec((N, D)), _rows_spec((N, D))],
    out_shape=[jax.ShapeDtypeStruct((N, D), jnp.float32),
               jax.ShapeDtypeStruct((N, D), jnp.float32)],
)

_tc_last = pl.pallas_call(
    _tc_last_body,
    grid=(GRID,),
    in_specs=[_rows_spec((N, D)), _rows_spec_hi((N, D)), _rows_spec((N, D)),
              _rows_spec((N, 1)), _full_b],
    out_specs=_rows_spec((N, D)),
    out_shape=jax.ShapeDtypeStruct((N, D), jnp.float32),
)


def kernel(x, edge_index, W0, b0, W1, b1, W2, b2):
    srcf = edge_index[0].astype(jnp.int32)
    dstf = edge_index[1].astype(jnp.int32)
    dst = dstf.reshape(NW, CPW, K)
    zeros = jnp.zeros((N, D), jnp.float32)
    zeros1 = jnp.zeros((N, DW), jnp.float32)
    ones = jnp.ones((K, DW), jnp.float32)

    deg2 = _degree(dst, zeros1, ones)
    dis, g0 = _tc0(x, W0, deg2, deg2)

    b0 = b0.reshape(1, D)
    b1 = b1.reshape(1, D)
    b2 = b2.reshape(1, D)

    acc0 = _gather_scatter(g0, srcf, dstf, zeros)
    x1, g1 = _tc_mid(acc0, acc0, g0, dis, b0, W1)
    acc1 = _gather_scatter(g1, srcf, dstf, zeros)
    x2, g2 = _tc_mid(acc1, acc1, g1, dis, b1, W2)
    acc2 = _gather_scatter(g2, srcf, dstf, zeros)
    x3 = _tc_last(acc2, acc2, g2, dis, b2)
    return jnp.concatenate([x, x1, x2, x3], axis=-1)


# speedup vs baseline: 1.3623x; 1.2292x over previous
import functools

import jax
import jax.numpy as jnp
from jax import lax
from jax.experimental import pallas as pl
from jax.experimental.pallas import tpu as pltpu
from jax.experimental.pallas import tpu_sc as plsc

N = 10000
E = 320000
D = 128
NC = 2
NS = 16
NW = NC * NS
K = 80
CPW = E // K // NW
RPT = 632
RPT_LAST = N - (NS - 1) * RPT
DW = 128
NSUB = 2
NBUF = 3
NI = 9
KH = K // NSUB
BLK = 1000
GRID = N // BLK

_mesh = plsc.VectorSubcoreMesh(
    core_axis_name="c", subcore_axis_name="s", num_cores=NC, num_subcores=NS)


def _degree_body(dst_hbm, zeros1_hbm, ones_hbm, out_hbm, deg_sp, dst_v, ones_v,
                 sem):
    c = lax.axis_index("c")
    s = lax.axis_index("s")
    wid = s * NC + c

    @pl.when(s == 0)
    def _():
        pltpu.sync_copy(zeros1_hbm, deg_sp)

    pltpu.sync_copy(dst_hbm.at[wid], dst_v)
    pltpu.sync_copy(ones_hbm, ones_v)
    plsc.subcore_barrier()

    FL = 5

    def body(i, carry):
        for u in range(FL):
            pltpu.async_copy(ones_v, deg_sp.at[dst_v.at[i * FL + u]], sem,
                             add=True)
        for u in range(FL):
            pltpu.make_async_copy(ones_v, deg_sp.at[dst_v.at[i * FL + u]],
                                  sem).wait()
        return carry

    lax.fori_loop(0, CPW // FL, body, 0)
    plsc.subcore_barrier()

    @pl.when(s == 0)
    def _():
        pltpu.sync_copy(deg_sp, out_hbm.at[pl.ds(c * N, N)])


def _gather_scatter_body(g_hbm, srcf_hbm, dstf_hbm, zeros_hbm, out_hbm,
                         acc_sp, src_st, dst_st, rows_v, gsems, isems):
    c = lax.axis_index("c")
    s = lax.axis_index("s")
    wid = s * NC + c
    ebase = wid * CPW * K

    @pl.when(s < NS - 1)
    def _():
        pltpu.sync_copy(zeros_hbm.at[pl.ds(s * RPT, RPT)],
                        acc_sp.at[pl.ds(s * RPT, RPT)])

    @pl.when(s == NS - 1)
    def _():
        pltpu.sync_copy(zeros_hbm.at[pl.ds((NS - 1) * RPT, RPT_LAST)],
                        acc_sp.at[pl.ds((NS - 1) * RPT, RPT_LAST)])

    def idx_load(j, b):
        pltpu.async_copy(srcf_hbm.at[pl.ds(ebase + j * K, K)], src_st[b],
                         isems[b])
        pltpu.async_copy(dstf_hbm.at[pl.ds(ebase + j * K, K)], dst_st[b],
                         isems[b])

    def idx_wait(j, b):
        pltpu.make_async_copy(srcf_hbm.at[pl.ds(ebase + j * K, K)], src_st[b],
                              isems[b]).wait()
        pltpu.make_async_copy(dstf_hbm.at[pl.ds(ebase + j * K, K)], dst_st[b],
                              isems[b]).wait()

    def g_issue(b, e):
        for u in range(NSUB):
            pltpu.async_copy(g_hbm.at[src_st[e].at[pl.ds(u * KH, KH)]],
                             rows_v[b].at[pl.ds(u * KH, KH)], gsems[b])

    def g_wait(b, e):
        for u in range(NSUB):
            pltpu.make_async_copy(g_hbm.at[src_st[e].at[pl.ds(u * KH, KH)]],
                                  rows_v[b].at[pl.ds(u * KH, KH)],
                                  gsems[b]).wait()

    for b in range(NI):
        idx_load(b, b)
    plsc.subcore_barrier()
    for b in range(2):
        idx_wait(b, b)
        g_issue(b, b)

    def step(j, b, e):
        g_wait(b, e)
        b2 = (b + 2) % NBUF
        e2 = (e + 2) % NI

        @pl.when(j + 2 < CPW)
        def _():
            idx_wait(j + 2, e2)
            g_issue(b2, e2)

        pltpu.sync_copy(rows_v[b], acc_sp.at[dst_st[e]], add=True)

        @pl.when(j + NI < CPW)
        def _():
            idx_load(j + NI, e)

    UN = 9

    def bodyn(i, carry):
        for u in range(UN):
            step(UN * i + u, u % NBUF, u % NI)
        return carry

    lax.fori_loop(0, CPW // UN, bodyn, 0)
    for u in range(CPW % UN):
        j = (CPW // UN) * UN + u
        step(j, j % NBUF, j % NI)
    plsc.subcore_barrier()

    @pl.when(s < NS - 1)
    def _():
        pltpu.sync_copy(acc_sp.at[pl.ds(s * RPT, RPT)],
                        out_hbm.at[pl.ds(c * N + s * RPT, RPT)])

    @pl.when(s == NS - 1)
    def _():
        pltpu.sync_copy(acc_sp.at[pl.ds((NS - 1) * RPT, RPT_LAST)],
                        out_hbm.at[pl.ds(c * N + (NS - 1) * RPT, RPT_LAST)])


def _make_degree(interpret=False):
    return pl.kernel(
        _degree_body,
        out_type=jax.ShapeDtypeStruct((NC * N, DW), jnp.float32),
        mesh=_mesh,
        scratch_types=[
            pltpu.VMEM_SHARED((N, DW), jnp.float32),
            pltpu.VMEM((CPW, K), jnp.int32),
            pltpu.VMEM((K, DW), jnp.float32),
            pltpu.SemaphoreType.DMA,
        ],
        interpret=interpret,
    )


def _make_gather_scatter(interpret=False):
    return pl.kernel(
        _gather_scatter_body,
        out_type=jax.ShapeDtypeStruct((NC * N, D), jnp.float32),
        mesh=_mesh,
        scratch_types=[
            pltpu.VMEM_SHARED((N, D), jnp.float32),
            [pltpu.VMEM((K,), jnp.int32) for _ in range(NI)],
            [pltpu.VMEM((K,), jnp.int32) for _ in range(NI)],
            [pltpu.VMEM((K, D), jnp.float32) for _ in range(NBUF)],
            [pltpu.SemaphoreType.DMA for _ in range(NBUF)],
            [pltpu.SemaphoreType.DMA for _ in range(NI)],
        ],
        interpret=interpret,
    )


_degree = _make_degree()
_gather_scatter = _make_gather_scatter()


def _tc0_body(x_ref, w_ref, dega_ref, degb_ref, dis_ref, g_ref):
    deg = dega_ref[:, 0:1] + degb_ref[:, 0:1] + 1.0
    dis = lax.rsqrt(deg)
    dis_ref[...] = dis
    g_ref[...] = jnp.dot(x_ref[...], w_ref[...],
                         preferred_element_type=jnp.float32) * dis


def _tc_mid_body(acca_ref, accb_ref, g_ref, dis_ref, b_ref, w_ref,
                 x_ref, gout_ref):
    dis = dis_ref[...]
    xl = dis * (acca_ref[...] + accb_ref[...] + g_ref[...]) + b_ref[...]
    x_ref[...] = xl
    gout_ref[...] = jnp.dot(xl, w_ref[...],
                            preferred_element_type=jnp.float32) * dis


def _tc_last_body(acca_ref, accb_ref, g_ref, dis_ref, b_ref, x_ref):
    x_ref[...] = (dis_ref[...] * (acca_ref[...] + accb_ref[...] + g_ref[...])
                  + b_ref[...])


def _rows_spec(shape):
    return pl.BlockSpec((BLK,) + shape[1:], lambda i: (i,) + (0,) * (len(shape) - 1))


def _rows_spec_hi(shape):
    return pl.BlockSpec((BLK,) + shape[1:],
                        lambda i: (i + GRID,) + (0,) * (len(shape) - 1))


_full_w = pl.BlockSpec((D, D), lambda i: (0, 0))
_full_b = pl.BlockSpec((1, D), lambda i: (0, 0))

_tc0 = pl.pallas_call(
    _tc0_body,
    grid=(GRID,),
    in_specs=[_rows_spec((N, D)), _full_w,
              _rows_spec((N, DW)), _rows_spec_hi((N, DW))],
    out_specs=[_rows_spec((N, 1)), _rows_spec((N, D))],
    out_shape=[jax.ShapeDtypeStruct((N, 1), jnp.float32),
               jax.ShapeDtypeStruct((N, D), jnp.float32)],
)

_tc_mid = pl.pallas_call(
    _tc_mid_body,
    grid=(GRID,),
    in_specs=[_rows_spec((N, D)), _rows_spec_hi((N, D)), _rows_spec((N, D)),
              _rows_spec((N, 1)), _full_b, _full_w],
    out_specs=[_rows_spec((N, D)), _rows_spec((N, D))],
    out_shape=[jax.ShapeDtypeStruct((N, D), jnp.float32),
               jax.ShapeDtypeStruct((N, D), jnp.float32)],
)

_tc_last = pl.pallas_call(
    _tc_last_body,
    grid=(GRID,),
    in_specs=[_rows_spec((N, D)), _rows_spec_hi((N, D)), _rows_spec((N, D)),
              _rows_spec((N, 1)), _full_b],
    out_specs=_rows_spec((N, D)),
    out_shape=jax.ShapeDtypeStruct((N, D), jnp.float32),
)


def kernel(x, edge_index, W0, b0, W1, b1, W2, b2):
    srcf = edge_index[0].astype(jnp.int32)
    dstf = edge_index[1].astype(jnp.int32)
    dst = dstf.reshape(NW, CPW, K)
    zeros = jnp.zeros((N, D), jnp.float32)
    zeros1 = jnp.zeros((N, DW), jnp.float32)
    ones = jnp.ones((K, DW), jnp.float32)

    deg2 = _degree(dst, zeros1, ones)
    dis, g0 = _tc0(x, W0, deg2, deg2)

    b0 = b0.reshape(1, D)
    b1 = b1.reshape(1, D)
    b2 = b2.reshape(1, D)

    acc0 = _gather_scatter(g0, srcf, dstf, zeros)
    x1, g1 = _tc_mid(acc0, acc0, g0, dis, b0, W1)
    acc1 = _gather_scatter(g1, srcf, dstf, zeros)
    x2, g2 = _tc_mid(acc1, acc1, g1, dis, b1, W2)
    acc2 = _gather_scatter(g2, srcf, dstf, zeros)
    x3 = _tc_last(acc2, acc2, g2, dis, b2)
    return jnp.concatenate([x, x1, x2, x3], axis=-1)
